# baseline (speedup 1.0000x reference)
import functools

import jax
import jax.numpy as jnp
from jax import lax
from jax.experimental import pallas as pl
from jax.experimental.pallas import tpu as pltpu
from jax.experimental.pallas import tpu_sc as plsc

N = 10000
E = 320000
DH = 256
NW = 32
RPW = 320
NPAD = NW * RPW
ACCR = 336
CAP = 12288
PADB = 64
SEG = CAP + PADB
CH = 4000
BATCH = 64


def _sc_mesh():
    return plsc.VectorSubcoreMesh(core_axis_name="c", subcore_axis_name="s")


_SC_PARAMS = pltpu.CompilerParams(needs_layout_passes=False)


def _wid():
    return lax.axis_index("s") * 2 + lax.axis_index("c")


def _part_body(row_hbm, col_hbm, pk_hbm, counts_hbm, deg_hbm,
               colv, rowv, erows, elc, packed, hist, cntbuf):
    w = _wid()
    lo = w * RPW
    zero16 = jnp.zeros((16,), jnp.float32)
    one16 = jnp.ones((16,), jnp.float32)

    def zstep(i, _):
        hist[pl.ds(i * 16, 16)] = zero16
        return 0

    lax.fori_loop(0, ACCR // 16, zstep, 0)

    def chunk(ch, cnt):
        pltpu.sync_copy(row_hbm.at[pl.ds(ch * CH, CH)], rowv)
        pltpu.sync_copy(col_hbm.at[pl.ds(ch * CH, CH)], colv)

        def cstep(i, c):
            cvs = [colv[pl.ds(i * 80 + t * 16, 16)] for t in range(5)]
            rvs = [rowv[pl.ds(i * 80 + t * 16, 16)] for t in range(5)]
            lcs = [cv - lo for cv in cvs]
            ms = [(lc >= 0) & (lc < RPW) for lc in lcs]
            ps = [plsc.all_reduce_population_count(m)[0] for m in ms]
            for t in range(5):
                plsc.store_compressed(erows.at[pl.ds(c, 16)], rvs[t],
                                      mask=ms[t])
                plsc.store_compressed(elc.at[pl.ds(c, 16)], lcs[t],
                                      mask=ms[t])
                c = jnp.minimum(c + ps[t], CAP)
            return c

        return lax.fori_loop(0, CH // 80, cstep, cnt)

    cnt = lax.fori_loop(0, E // CH, chunk, jnp.int32(0))

    pad_r = jnp.zeros((16,), jnp.int32)
    pad_c = jnp.full((16,), RPW, jnp.int32)
    for j in range(PADB // 16):
        erows[pl.ds(cnt + j * 16, 16)] = pad_r
        elc[pl.ds(cnt + j * 16, 16)] = pad_c

    def hstep(i, _):
        lcs = elc[pl.ds(i * 16, 16)]
        plsc.addupdate_scatter(hist, [lcs], one16)
        return 0

    lax.fori_loop(0, (cnt + 15) // 16, hstep, 0)

    cntbuf[pl.ds(0, 16)] = jnp.full((16,), cnt, jnp.int32)

    def pstep(b, _):
        for j in range(BATCH // 16):
            packed[pl.ds(b * 2 * BATCH + j * 16, 16)] = (
                erows[pl.ds(b * BATCH + j * 16, 16)])
            packed[pl.ds(b * 2 * BATCH + BATCH + j * 16, 16)] = (
                elc[pl.ds(b * BATCH + j * 16, 16)])
        return 0

    lax.fori_loop(0, SEG // BATCH, pstep, 0)

    pltpu.sync_copy(packed, pk_hbm.at[pl.ds(w * 2 * SEG, 2 * SEG)])
    pltpu.sync_copy(cntbuf, counts_hbm.at[pl.ds(w * 16, 16)])
    pltpu.sync_copy(hist.at[pl.ds(0, RPW)], deg_hbm.at[pl.ds(w * RPW, RPW)])


def _sc_partition(row, col):
    f = pl.kernel(
        _part_body,
        out_type=(
            jax.ShapeDtypeStruct((NW * 2 * SEG,), jnp.int32),
            jax.ShapeDtypeStruct((NW * 16,), jnp.int32),
            jax.ShapeDtypeStruct((NPAD,), jnp.float32),
        ),
        mesh=_sc_mesh(),
        compiler_params=_SC_PARAMS,
        scratch_types=[
            pltpu.VMEM((CH,), jnp.int32),
            pltpu.VMEM((CH,), jnp.int32),
            pltpu.VMEM((SEG,), jnp.int32),
            pltpu.VMEM((SEG,), jnp.int32),
            pltpu.VMEM((2 * SEG,), jnp.int32),
            pltpu.VMEM((ACCR,), jnp.float32),
            pltpu.VMEM((16,), jnp.int32),
        ],
    )
    return f(row, col)


_PIPELINED = True


def _agg_body(g_hbm, pk_hbm, counts_hbm, out_hbm,
              cntv, pk0, pk1, st0, st1, acc,
              semg0, semg1, semi0, semi1):
    w = _wid()
    base = w * 2 * SEG
    zero16 = jnp.zeros((16,), jnp.float32)

    def zrow(r, _):
        for j in range(DH // 16):
            acc[r, pl.ds(j * 16, 16)] = zero16
        return 0

    lax.fori_loop(0, ACCR, zrow, 0)

    pltpu.sync_copy(counts_hbm.at[pl.ds(w * 16, 16)], cntv)
    cnt = cntv[pl.ds(0, 16)][0]
    nb = (cnt + BATCH - 1) // BATCH

    def pk_src(b):
        return pk_hbm.at[pl.ds(base + b * 2 * BATCH, 2 * BATCH)]

    if _PIPELINED:
        @pl.when(nb > 0)
        def _():
            pltpu.sync_copy(pk_src(0), pk0)
            pltpu.async_copy(g_hbm.at[pk0.at[pl.ds(0, BATCH)]], st0, semg0)

        @pl.when(nb > 1)
        def _():
            pltpu.async_copy(pk_src(1), pk1, semi1)

    def step(b, mypk, myst, mysemg, myisem, otpk, otst, otsemg, otisem):
        @pl.when(b + 1 < nb)
        def _():
            pltpu.make_async_copy(pk_src(b + 1), otpk, otisem).wait()
            pltpu.async_copy(g_hbm.at[otpk.at[pl.ds(0, BATCH)]], otst, otsemg)

        pltpu.make_async_copy(
            g_hbm.at[mypk.at[pl.ds(0, BATCH)]], myst, mysemg
        ).wait()

        def load_row(r):
            return [myst[r, pl.ds(j * 16, 16)] for j in range(DH // 16)]

        def rstep(i, vals):
            lcs = mypk[pl.ds(BATCH + i * 16, 16)]
            for k in range(16):
                lc = lcs[k]
                nxt = (i * 16 + k + 1) & (BATCH - 1)
                nvals = load_row(nxt)
                for j in range(DH // 16):
                    plsc.addupdate(acc.at[lc, pl.ds(j * 16, 16)], vals[j])
                vals = nvals
            return vals

        lax.fori_loop(0, BATCH // 16, rstep, load_row(0))

        @pl.when(b + 2 < nb)
        def _():
            pltpu.async_copy(pk_src(b + 2), mypk, myisem)

    def bstep_sync(b, _):
        pltpu.sync_copy(pk_src(b), pk0)
        pltpu.async_copy(g_hbm.at[pk0.at[pl.ds(0, BATCH)]], st0, semg0).wait()

        def rstep(i, _):
            lcs = pk0[pl.ds(BATCH + i * 16, 16)]
            for k in range(16):
                lc = lcs[k]
                r = i * 16 + k
                for j in range(DH // 16):
                    plsc.addupdate(acc.at[lc, pl.ds(j * 16, 16)],
                                   st0[r, pl.ds(j * 16, 16)])
            return 0

        lax.fori_loop(0, BATCH // 16, rstep, 0)
        return 0

    def bstep(b, _):
        @pl.when(b % 2 == 0)
        def _():
            step(b, pk0, st0, semg0, semi0, pk1, st1, semg1, semi1)

        @pl.when(b % 2 == 1)
        def _():
            step(b, pk1, st1, semg1, semi1, pk0, st0, semg0, semi0)

        return 0

    if not _PIPELINED:
        bstep = bstep_sync

    lax.fori_loop(0, nb, bstep, 0)

    pltpu.sync_copy(acc.at[pl.ds(0, RPW)], out_hbm.at[pl.ds(w * RPW, RPW)])


def _sc_agg(g, pk, counts):
    f = pl.kernel(
        _agg_body,
        out_type=jax.ShapeDtypeStruct((NPAD, DH), jnp.float32),
        mesh=_sc_mesh(),
        compiler_params=_SC_PARAMS,
        scratch_types=[
            pltpu.VMEM((16,), jnp.int32),
            pltpu.VMEM((2 * BATCH,), jnp.int32),
            pltpu.VMEM((2 * BATCH,), jnp.int32),
            pltpu.VMEM((BATCH, DH), jnp.float32),
            pltpu.VMEM((BATCH, DH), jnp.float32),
            pltpu.VMEM((ACCR, DH), jnp.float32),
            pltpu.SemaphoreType.DMA,
            pltpu.SemaphoreType.DMA,
            pltpu.SemaphoreType.DMA,
            pltpu.SemaphoreType.DMA,
        ],
    )
    return f(g, pk, counts)[:N]


def _tc1_body(deg_ref, x_ref, w_ref, o_ref):
    dv = lax.rsqrt(deg_ref[...] + 1.0)
    o_ref[...] = jnp.dot(x_ref[...] * dv, w_ref[...],
                         preferred_element_type=jnp.float32)


def _tc1(deg, x, W1):
    return pl.pallas_call(
        _tc1_body,
        grid=(10,),
        in_specs=[
            pl.BlockSpec((1000, 1), lambda i: (i, 0)),
            pl.BlockSpec((1000, 128), lambda i: (i, 0)),
            pl.BlockSpec((128, DH), lambda i: (0, 0)),
        ],
        out_specs=pl.BlockSpec((1000, DH), lambda i: (i, 0)),
        out_shape=jax.ShapeDtypeStruct((N, DH), jnp.float32),
    )(deg, x, W1)


def _tc23_body(deg_ref, acc_ref, g_ref, w_ref, b_ref, o_ref, *, post_scale):
    dv = lax.rsqrt(deg_ref[...] + 1.0)
    z = jnp.maximum(dv * (acc_ref[...] + g_ref[...]) + b_ref[...], 0.0)
    if post_scale:
        z = z * dv
    o_ref[...] = jnp.dot(z, w_ref[...], preferred_element_type=jnp.float32)


def _tc23(deg, acc, g, W, b, n_out, post_scale):
    body = functools.partial(_tc23_body, post_scale=post_scale)
    return pl.pallas_call(
        body,
        grid=(10,),
        in_specs=[
            pl.BlockSpec((1000, 1), lambda i: (i, 0)),
            pl.BlockSpec((1000, DH), lambda i: (i, 0)),
            pl.BlockSpec((1000, DH), lambda i: (i, 0)),
            pl.BlockSpec((DH, n_out), lambda i: (0, 0)),
            pl.BlockSpec((1, DH), lambda i: (0, 0)),
        ],
        out_specs=pl.BlockSpec((1000, n_out), lambda i: (i, 0)),
        out_shape=jax.ShapeDtypeStruct((N, n_out), jnp.float32),
    )(deg, acc, g, W, b)


def kernel(x, edge_index, W1, b1, W2, b2, Wout, bout):
    row = edge_index[0]
    col = edge_index[1]
    pk, counts, degp = _sc_partition(row, col)
    deg = degp[:N].reshape(N, 1)
    g1 = _tc1(deg, x, W1)
    acc1 = _sc_agg(g1, pk, counts)
    g2 = _tc23(deg, acc1, g1, W2, b1.reshape(1, DH), DH, post_scale=True)
    acc2 = _sc_agg(g2, pk, counts)
    Wp = jnp.zeros((DH, 128), jnp.float32).at[:, :4].set(Wout)
    outp = _tc23(deg, acc2, g2, Wp, b2.reshape(1, DH), 128, post_scale=False)
    return outp[:, :4] + bout


# speedup vs baseline: 1.0081x; 1.0081x over previous
import functools

import jax
import jax.numpy as jnp
from jax import lax
from jax.experimental import pallas as pl
from jax.experimental.pallas import tpu as pltpu
from jax.experimental.pallas import tpu_sc as plsc

N = 10000
E = 320000
DH = 256
NW = 32
RPW = 320
NPAD = NW * RPW
ACCR = 336
CAP = 12288
PADB = 64
SEG = CAP + PADB
CH = 4000
BATCH = 64


def _sc_mesh():
    return plsc.VectorSubcoreMesh(core_axis_name="c", subcore_axis_name="s")


_SC_PARAMS = pltpu.CompilerParams(needs_layout_passes=False)


def _wid():
    return lax.axis_index("s") * 2 + lax.axis_index("c")


def _part_body(row_hbm, col_hbm, pk_hbm, counts_hbm, deg_hbm,
               colv, rowv, erows, elc, packed, hist, cntbuf):
    w = _wid()
    lo = w * RPW
    zero16 = jnp.zeros((16,), jnp.float32)
    one16 = jnp.ones((16,), jnp.float32)

    def zstep(i, _):
        hist[pl.ds(i * 16, 16)] = zero16
        return 0

    lax.fori_loop(0, ACCR // 16, zstep, 0)

    def chunk(ch, cnt):
        pltpu.sync_copy(row_hbm.at[pl.ds(ch * CH, CH)], rowv)
        pltpu.sync_copy(col_hbm.at[pl.ds(ch * CH, CH)], colv)

        def cstep(i, c):
            cvs = [colv[pl.ds(i * 80 + t * 16, 16)] for t in range(5)]
            rvs = [rowv[pl.ds(i * 80 + t * 16, 16)] for t in range(5)]
            lcs = [cv - lo for cv in cvs]
            ms = [(lc >= 0) & (lc < RPW) for lc in lcs]
            ps = [plsc.all_reduce_population_count(m)[0] for m in ms]
            for t in range(5):
                plsc.store_compressed(erows.at[pl.ds(c, 16)], rvs[t],
                                      mask=ms[t])
                plsc.store_compressed(elc.at[pl.ds(c, 16)], lcs[t],
                                      mask=ms[t])
                c = jnp.minimum(c + ps[t], CAP)
            return c

        return lax.fori_loop(0, CH // 80, cstep, cnt)

    cnt = lax.fori_loop(0, E // CH, chunk, jnp.int32(0))

    pad_r = jnp.zeros((16,), jnp.int32)
    pad_c = jnp.full((16,), RPW, jnp.int32)
    for j in range(PADB // 16):
        erows[pl.ds(cnt + j * 16, 16)] = pad_r
        elc[pl.ds(cnt + j * 16, 16)] = pad_c

    def hstep(i, _):
        lcs = elc[pl.ds(i * 16, 16)]
        plsc.addupdate_scatter(hist, [lcs], one16)
        return 0

    lax.fori_loop(0, (cnt + 15) // 16, hstep, 0)

    cntbuf[pl.ds(0, 16)] = jnp.full((16,), cnt, jnp.int32)

    def pstep(b, _):
        for j in range(BATCH // 16):
            packed[pl.ds(b * 2 * BATCH + j * 16, 16)] = (
                erows[pl.ds(b * BATCH + j * 16, 16)])
            packed[pl.ds(b * 2 * BATCH + BATCH + j * 16, 16)] = (
                elc[pl.ds(b * BATCH + j * 16, 16)])
        return 0

    lax.fori_loop(0, SEG // BATCH, pstep, 0)

    pltpu.sync_copy(packed, pk_hbm.at[pl.ds(w * 2 * SEG, 2 * SEG)])
    pltpu.sync_copy(cntbuf, counts_hbm.at[pl.ds(w * 16, 16)])
    pltpu.sync_copy(hist.at[pl.ds(0, RPW)], deg_hbm.at[pl.ds(w * RPW, RPW)])


def _sc_partition(row, col):
    f = pl.kernel(
        _part_body,
        out_type=(
            jax.ShapeDtypeStruct((NW * 2 * SEG,), jnp.int32),
            jax.ShapeDtypeStruct((NW * 16,), jnp.int32),
            jax.ShapeDtypeStruct((NPAD,), jnp.float32),
        ),
        mesh=_sc_mesh(),
        compiler_params=_SC_PARAMS,
        scratch_types=[
            pltpu.VMEM((CH,), jnp.int32),
            pltpu.VMEM((CH,), jnp.int32),
            pltpu.VMEM((SEG,), jnp.int32),
            pltpu.VMEM((SEG,), jnp.int32),
            pltpu.VMEM((2 * SEG,), jnp.int32),
            pltpu.VMEM((ACCR,), jnp.float32),
            pltpu.VMEM((16,), jnp.int32),
        ],
    )
    return f(row, col)


def _agg_body(g_hbm, pk_hbm, counts_hbm, out_hbm,
              cntv, pk0, pk1, st0, st1, acc,
              semg0, semg1, semi0, semi1):
    w = _wid()
    base = w * 2 * SEG
    zero16 = jnp.zeros((16,), jnp.float32)

    def zrow(r, _):
        for j in range(DH // 16):
            acc[r, pl.ds(j * 16, 16)] = zero16
        return 0

    lax.fori_loop(0, ACCR, zrow, 0)

    pltpu.sync_copy(counts_hbm.at[pl.ds(w * 16, 16)], cntv)
    cnt = cntv[pl.ds(0, 16)][0]
    nb = (cnt + BATCH - 1) // BATCH

    def pk_src(b):
        return pk_hbm.at[pl.ds(base + b * 2 * BATCH, 2 * BATCH)]

    @pl.when(nb > 0)
    def _():
        pltpu.sync_copy(pk_src(0), pk0)
        pltpu.async_copy(g_hbm.at[pk0.at[pl.ds(0, BATCH)]], st0, semg0)

    @pl.when(nb > 1)
    def _():
        pltpu.async_copy(pk_src(1), pk1, semi1)

    def step(b, mypk, myst, mysemg, myisem, otpk, otst, otsemg, otisem):
        @pl.when(b + 1 < nb)
        def _():
            pltpu.make_async_copy(pk_src(b + 1), otpk, otisem).wait()
            pltpu.async_copy(g_hbm.at[otpk.at[pl.ds(0, BATCH)]], otst, otsemg)

        pltpu.make_async_copy(
            g_hbm.at[mypk.at[pl.ds(0, BATCH)]], myst, mysemg
        ).wait()

        def rstep(i, _):
            lcs = mypk[pl.ds(BATCH + i * 16, 16)]
            vals = [myst[i * 16, pl.ds(j * 16, 16)] for j in range(DH // 16)]
            for k in range(16):
                lc = lcs[k]
                if k < 15:
                    nvals = [myst[i * 16 + k + 1, pl.ds(j * 16, 16)]
                             for j in range(DH // 16)]
                for j in range(DH // 16):
                    plsc.addupdate(acc.at[lc, pl.ds(j * 16, 16)], vals[j])
                if k < 15:
                    vals = nvals
            return 0

        lax.fori_loop(0, BATCH // 16, rstep, 0)

        @pl.when(b + 2 < nb)
        def _():
            pltpu.async_copy(pk_src(b + 2), mypk, myisem)

    def bstep(b, _):
        @pl.when(b % 2 == 0)
        def _():
            step(b, pk0, st0, semg0, semi0, pk1, st1, semg1, semi1)

        @pl.when(b % 2 == 1)
        def _():
            step(b, pk1, st1, semg1, semi1, pk0, st0, semg0, semi0)

        return 0

    lax.fori_loop(0, nb, bstep, 0)

    pltpu.sync_copy(acc.at[pl.ds(0, RPW)], out_hbm.at[pl.ds(w * RPW, RPW)])


def _sc_agg(g, pk, counts):
    f = pl.kernel(
        _agg_body,
        out_type=jax.ShapeDtypeStruct((NPAD, DH), jnp.float32),
        mesh=_sc_mesh(),
        compiler_params=_SC_PARAMS,
        scratch_types=[
            pltpu.VMEM((16,), jnp.int32),
            pltpu.VMEM((2 * BATCH,), jnp.int32),
            pltpu.VMEM((2 * BATCH,), jnp.int32),
            pltpu.VMEM((BATCH, DH), jnp.float32),
            pltpu.VMEM((BATCH, DH), jnp.float32),
            pltpu.VMEM((ACCR, DH), jnp.float32),
            pltpu.SemaphoreType.DMA,
            pltpu.SemaphoreType.DMA,
            pltpu.SemaphoreType.DMA,
            pltpu.SemaphoreType.DMA,
        ],
    )
    return f(g, pk, counts)[:N]


def _tc1_body(deg_ref, x_ref, w_ref, o_ref):
    dv = lax.rsqrt(deg_ref[...] + 1.0)
    o_ref[...] = jnp.dot(x_ref[...] * dv, w_ref[...],
                         preferred_element_type=jnp.float32)


def _tc1(deg, x, W1):
    return pl.pallas_call(
        _tc1_body,
        grid=(10,),
        in_specs=[
            pl.BlockSpec((1000, 1), lambda i: (i, 0)),
            pl.BlockSpec((1000, 128), lambda i: (i, 0)),
            pl.BlockSpec((128, DH), lambda i: (0, 0)),
        ],
        out_specs=pl.BlockSpec((1000, DH), lambda i: (i, 0)),
        out_shape=jax.ShapeDtypeStruct((N, DH), jnp.float32),
    )(deg, x, W1)


def _tc23_body(deg_ref, acc_ref, g_ref, w_ref, b_ref, o_ref, *, post_scale):
    dv = lax.rsqrt(deg_ref[...] + 1.0)
    z = jnp.maximum(dv * (acc_ref[...] + g_ref[...]) + b_ref[...], 0.0)
    if post_scale:
        z = z * dv
    o_ref[...] = jnp.dot(z, w_ref[...], preferred_element_type=jnp.float32)


def _tc23(deg, acc, g, W, b, n_out, post_scale):
    body = functools.partial(_tc23_body, post_scale=post_scale)
    return pl.pallas_call(
        body,
        grid=(10,),
        in_specs=[
            pl.BlockSpec((1000, 1), lambda i: (i, 0)),
            pl.BlockSpec((1000, DH), lambda i: (i, 0)),
            pl.BlockSpec((1000, DH), lambda i: (i, 0)),
            pl.BlockSpec((DH, n_out), lambda i: (0, 0)),
            pl.BlockSpec((1, DH), lambda i: (0, 0)),
        ],
        out_specs=pl.BlockSpec((1000, n_out), lambda i: (i, 0)),
        out_shape=jax.ShapeDtypeStruct((N, n_out), jnp.float32),
    )(deg, acc, g, W, b)


def kernel(x, edge_index, W1, b1, W2, b2, Wout, bout):
    row = edge_index[0]
    col = edge_index[1]
    pk, counts, degp = _sc_partition(row, col)
    deg = degp[:N].reshape(N, 1)
    g1 = _tc1(deg, x, W1)
    acc1 = _sc_agg(g1, pk, counts)
    g2 = _tc23(deg, acc1, g1, W2, b1.reshape(1, DH), DH, post_scale=True)
    acc2 = _sc_agg(g2, pk, counts)
    Wp = jnp.zeros((DH, 128), jnp.float32).at[:, :4].set(Wout)
    outp = _tc23(deg, acc2, g2, Wp, b2.reshape(1, DH), 128, post_scale=False)
    return outp[:, :4] + bout
